# baseline (device time: 173700 ns/iter reference)
import functools

import numpy as np
import jax
import jax.numpy as jnp
from jax import lax
from jax.experimental import pallas as pl
from jax.experimental.pallas import tpu as pltpu

N_DEV = 16
B, SQ, D = 2, 256, 768
HQ_LOCAL, DH = 4, 64


def _rope_tables():
    inv = 1.0 / (10000.0 ** (np.arange(0, DH, 2) / DH))
    pos = np.arange(SQ)[:, None] * inv[None, :]
    cosv = np.cos(pos).astype(np.float32)
    sinv = np.sin(pos).astype(np.float32)
    cos_d = np.concatenate([cosv, cosv], axis=1)
    sin_d = np.concatenate([sinv, sinv], axis=1)
    return jnp.asarray(cos_d), jnp.asarray(sin_d)


def _deinterleave_cols(w):
    return w.reshape(D, HQ_LOCAL, DH // 2, 2).transpose(0, 1, 3, 2).reshape(D, HQ_LOCAL * DH)


def kernel(x, Wq, Wk, Wv, Wo):
    cos_d, sin_d = _rope_tables()
    xb = x.astype(jnp.bfloat16)
    wq = _deinterleave_cols(Wq).astype(jnp.bfloat16)
    wk = _deinterleave_cols(Wk).astype(jnp.bfloat16)
    wv = Wv.astype(jnp.bfloat16)
    wo = Wo.astype(jnp.bfloat16)

    def body(x_ref, wq_ref, wk_ref, wv_ref, wo_ref, cos_ref, sin_ref,
             out_ref, comm_ref, send_sems, recv_sems):
        my = lax.axis_index("i")
        left = lax.rem(my - 1 + N_DEV, N_DEV)
        right = lax.rem(my + 1, N_DEV)

        barrier_sem = pltpu.get_barrier_semaphore()
        for nbr in (left, right):
            pl.semaphore_signal(
                barrier_sem, inc=1,
                device_id=(nbr,), device_id_type=pl.DeviceIdType.MESH,
            )
        pl.semaphore_wait(barrier_sem, 2)

        cos = cos_ref[...]
        sin = sin_ref[...]
        for b in range(B):
            x_b = x_ref[b, :, :]
            acc = jnp.zeros((SQ, D), dtype=jnp.float32)
            for h in range(HQ_LOCAL):
                sl = slice(h * DH, (h + 1) * DH)
                q = jnp.dot(x_b, wq_ref[:, sl],
                            preferred_element_type=jnp.float32)
                k = jnp.dot(x_b, wk_ref[:, sl],
                            preferred_element_type=jnp.float32)
                v = jnp.dot(x_b, wv_ref[:, sl],
                            preferred_element_type=jnp.float32)

                def rope(t):
                    t_r = jnp.concatenate(
                        [-t[:, DH // 2:], t[:, :DH // 2]], axis=1)
                    return t * cos + t_r * sin

                q = rope(q).astype(jnp.bfloat16)
                k = rope(k).astype(jnp.bfloat16)
                s = lax.dot_general(
                    q, k, (((1,), (1,)), ((), ())),
                    preferred_element_type=jnp.float32) * 0.125
                m = jnp.max(s, axis=1, keepdims=True)
                e = jnp.exp(s - m)
                w = (e / jnp.sum(e, axis=1, keepdims=True)).astype(jnp.bfloat16)
                ctx = jnp.dot(w, v.astype(jnp.bfloat16),
                              preferred_element_type=jnp.float32)
                acc = acc + jnp.dot(ctx.astype(jnp.bfloat16), wo_ref[sl, :],
                                    preferred_element_type=jnp.float32)
            comm_ref[0, b, :, :] = acc.astype(jnp.bfloat16)

        for h in range(N_DEV - 1):
            rdma = pltpu.make_async_remote_copy(
                src_ref=comm_ref.at[h],
                dst_ref=comm_ref.at[h + 1],
                send_sem=send_sems.at[h],
                recv_sem=recv_sems.at[h],
                device_id=(right,),
                device_id_type=pl.DeviceIdType.MESH,
            )
            rdma.start()
            rdma.wait()

        for b in range(B):
            tot = comm_ref[0, b, :, :].astype(jnp.float32)
            for s in range(1, N_DEV):
                tot = tot + comm_ref[s, b, :, :].astype(jnp.float32)
            out_ref[b, :, :] = tot

    return pl.pallas_call(
        body,
        out_shape=jax.ShapeDtypeStruct((B, SQ, D), jnp.float32),
        in_specs=[pl.BlockSpec(memory_space=pltpu.VMEM)] * 7,
        out_specs=pl.BlockSpec(memory_space=pltpu.VMEM),
        scratch_shapes=[
            pltpu.VMEM((N_DEV, B, SQ, D), jnp.bfloat16),
            pltpu.SemaphoreType.DMA((N_DEV - 1,)),
            pltpu.SemaphoreType.DMA((N_DEV - 1,)),
        ],
        compiler_params=pltpu.CompilerParams(collective_id=0),
    )(xb, wq, wk, wv, wo, cos_d, sin_d)


# device time: 49803 ns/iter; 3.4877x vs baseline; 3.4877x over previous
import numpy as np
import jax
import jax.numpy as jnp
from jax import lax
from jax.experimental import pallas as pl
from jax.experimental.pallas import tpu as pltpu

N_DEV = 16
N_STAGES = 4
B, SQ, D = 2, 256, 768
ROWS = B * SQ
HQ_LOCAL, DH = 4, 64


def _rope_tables():
    inv = 1.0 / (10000.0 ** (np.arange(0, DH, 2) / DH))
    pos = np.arange(SQ)[:, None] * inv[None, :]
    cosv = np.cos(pos).astype(np.float32)
    sinv = np.sin(pos).astype(np.float32)
    cos_d = np.concatenate([cosv, cosv], axis=1)
    sin_d = np.concatenate([sinv, sinv], axis=1)
    return jnp.asarray(cos_d), jnp.asarray(sin_d)


def _deinterleave_cols(w):
    return w.reshape(D, HQ_LOCAL, DH // 2, 2).transpose(0, 1, 3, 2).reshape(D, HQ_LOCAL * DH)


def kernel(x, Wq, Wk, Wv, Wo):
    cos_d, sin_d = _rope_tables()
    xb = x.astype(jnp.bfloat16)
    wq = _deinterleave_cols(Wq).astype(jnp.bfloat16)
    wk = _deinterleave_cols(Wk).astype(jnp.bfloat16)
    wv = Wv.astype(jnp.bfloat16)
    wo = Wo.astype(jnp.bfloat16)

    def body(x_ref, wq_ref, wk_ref, wv_ref, wo_ref, cos_ref, sin_ref,
             out_ref, work_ref, rs0, rs1, rs2, rs3,
             rs_send, rs_recv, ag_send, ag_recv):
        rs_bufs = [rs0, rs1, rs2, rs3]
        my = lax.axis_index("i")

        barrier_sem = pltpu.get_barrier_semaphore()
        for s in range(N_STAGES):
            pl.semaphore_signal(
                barrier_sem, inc=1,
                device_id=(my ^ (1 << s),),
                device_id_type=pl.DeviceIdType.MESH,
            )
        pl.semaphore_wait(barrier_sem, N_STAGES)

        cos = cos_ref[...]
        sin = sin_ref[...]
        for b in range(B):
            x_b = x_ref[b, :, :]
            acc = jnp.zeros((SQ, D), dtype=jnp.float32)
            for h in range(HQ_LOCAL):
                sl = slice(h * DH, (h + 1) * DH)
                q = jnp.dot(x_b, wq_ref[:, sl],
                            preferred_element_type=jnp.float32)
                k = jnp.dot(x_b, wk_ref[:, sl],
                            preferred_element_type=jnp.float32)
                v = jnp.dot(x_b, wv_ref[:, sl],
                            preferred_element_type=jnp.float32)

                def rope(t):
                    t_r = jnp.concatenate(
                        [-t[:, DH // 2:], t[:, :DH // 2]], axis=1)
                    return t * cos + t_r * sin

                q = rope(q).astype(jnp.bfloat16)
                k = rope(k).astype(jnp.bfloat16)
                s_ = lax.dot_general(
                    q, k, (((1,), (1,)), ((), ())),
                    preferred_element_type=jnp.float32) * 0.125
                m = jnp.max(s_, axis=1, keepdims=True)
                e = jnp.exp(s_ - m)
                w = (e / jnp.sum(e, axis=1, keepdims=True)).astype(jnp.bfloat16)
                ctx = jnp.dot(w, v.astype(jnp.bfloat16),
                              preferred_element_type=jnp.float32)
                acc = acc + jnp.dot(ctx.astype(jnp.bfloat16), wo_ref[sl, :],
                                    preferred_element_type=jnp.float32)
            work_ref[b * SQ:(b + 1) * SQ, :] = acc.astype(jnp.bfloat16)

        off = jnp.int32(0)
        for s in range(N_STAGES):
            half = ROWS >> (s + 1)
            bit = (my >> s) & 1
            send_off = off + (1 - bit) * half
            keep_off = off + bit * half
            rdma = pltpu.make_async_remote_copy(
                src_ref=work_ref.at[pl.ds(send_off, half)],
                dst_ref=rs_bufs[s],
                send_sem=rs_send.at[s],
                recv_sem=rs_recv.at[s],
                device_id=(my ^ (1 << s),),
                device_id_type=pl.DeviceIdType.MESH,
            )
            rdma.start()
            rdma.wait()
            work_ref[pl.ds(keep_off, half), :] = (
                work_ref[pl.ds(keep_off, half), :].astype(jnp.float32)
                + rs_bufs[s][...].astype(jnp.float32)
            ).astype(jnp.bfloat16)
            off = keep_off

        cur = ROWS >> N_STAGES
        for s in reversed(range(N_STAGES)):
            rdma = pltpu.make_async_remote_copy(
                src_ref=work_ref.at[pl.ds(off, cur)],
                dst_ref=work_ref.at[pl.ds(off, cur)],
                send_sem=ag_send.at[s],
                recv_sem=ag_recv.at[s],
                device_id=(my ^ (1 << s),),
                device_id_type=pl.DeviceIdType.MESH,
            )
            rdma.start()
            rdma.wait()
            bit = (my >> s) & 1
            off = off - bit * cur
            cur *= 2

        for b in range(B):
            out_ref[b, :, :] = work_ref[b * SQ:(b + 1) * SQ, :].astype(jnp.float32)

    return pl.pallas_call(
        body,
        out_shape=jax.ShapeDtypeStruct((B, SQ, D), jnp.float32),
        in_specs=[pl.BlockSpec(memory_space=pltpu.VMEM)] * 7,
        out_specs=pl.BlockSpec(memory_space=pltpu.VMEM),
        scratch_shapes=[
            pltpu.VMEM((ROWS, D), jnp.bfloat16),
            pltpu.VMEM((ROWS >> 1, D), jnp.bfloat16),
            pltpu.VMEM((ROWS >> 2, D), jnp.bfloat16),
            pltpu.VMEM((ROWS >> 3, D), jnp.bfloat16),
            pltpu.VMEM((ROWS >> 4, D), jnp.bfloat16),
            pltpu.SemaphoreType.DMA((N_STAGES,)),
            pltpu.SemaphoreType.DMA((N_STAGES,)),
            pltpu.SemaphoreType.DMA((N_STAGES,)),
            pltpu.SemaphoreType.DMA((N_STAGES,)),
        ],
        compiler_params=pltpu.CompilerParams(collective_id=0),
    )(xb, wq, wk, wv, wo, cos_d, sin_d)


# device time: 37675 ns/iter; 4.6105x vs baseline; 1.3219x over previous
import numpy as np
import jax
import jax.numpy as jnp
from jax import lax
from jax.experimental import pallas as pl
from jax.experimental.pallas import tpu as pltpu

N_DEV = 16
B, SQ, D = 2, 256, 768
ROWS = B * SQ
CH = ROWS // N_DEV
HQ_LOCAL, DH = 4, 64


def _rope_tables():
    inv = 1.0 / (10000.0 ** (np.arange(0, DH, 2) / DH))
    pos = np.arange(SQ)[:, None] * inv[None, :]
    cosv = np.cos(pos).astype(np.float32)
    sinv = np.sin(pos).astype(np.float32)
    cos_d = np.concatenate([cosv, cosv], axis=1)
    sin_d = np.concatenate([sinv, sinv], axis=1)
    return jnp.asarray(cos_d), jnp.asarray(sin_d)


def _deinterleave_cols(w):
    return w.reshape(D, HQ_LOCAL, DH // 2, 2).transpose(0, 1, 3, 2).reshape(D, HQ_LOCAL * DH)


def kernel(x, Wq, Wk, Wv, Wo):
    cos_d, sin_d = _rope_tables()
    xb = x.astype(jnp.bfloat16)
    wq = _deinterleave_cols(Wq).astype(jnp.bfloat16)
    wk = _deinterleave_cols(Wk).astype(jnp.bfloat16)
    wv = Wv.astype(jnp.bfloat16)
    wo = Wo.astype(jnp.bfloat16)

    def body(x_ref, wq_ref, wk_ref, wv_ref, wo_ref, cos_ref, sin_ref,
             out_ref, work_ref, land_ref,
             rs_send, rs_recv, ag_send, ag_recv):
        my = lax.axis_index("i")

        barrier_sem = pltpu.get_barrier_semaphore()
        for j in range(N_DEV):
            @pl.when(j != my)
            def _():
                pl.semaphore_signal(
                    barrier_sem, inc=1,
                    device_id=(jnp.int32(j),),
                    device_id_type=pl.DeviceIdType.MESH,
                )
        pl.semaphore_wait(barrier_sem, N_DEV - 1)

        cos = cos_ref[...]
        sin = sin_ref[...]
        for b in range(B):
            x_b = x_ref[b, :, :]
            acc = jnp.zeros((SQ, D), dtype=jnp.float32)
            for h in range(HQ_LOCAL):
                sl = slice(h * DH, (h + 1) * DH)
                q = jnp.dot(x_b, wq_ref[:, sl],
                            preferred_element_type=jnp.float32)
                k = jnp.dot(x_b, wk_ref[:, sl],
                            preferred_element_type=jnp.float32)
                v = jnp.dot(x_b, wv_ref[:, sl],
                            preferred_element_type=jnp.float32)

                def rope(t):
                    t_r = jnp.concatenate(
                        [-t[:, DH // 2:], t[:, :DH // 2]], axis=1)
                    return t * cos + t_r * sin

                q = rope(q).astype(jnp.bfloat16)
                k = rope(k).astype(jnp.bfloat16)
                s_ = lax.dot_general(
                    q, k, (((1,), (1,)), ((), ())),
                    preferred_element_type=jnp.float32) * 0.125
                m = jnp.max(s_, axis=1, keepdims=True)
                e = jnp.exp(s_ - m)
                w = (e / jnp.sum(e, axis=1, keepdims=True)).astype(jnp.bfloat16)
                ctx = jnp.dot(w, v.astype(jnp.bfloat16),
                              preferred_element_type=jnp.float32)
                acc = acc + jnp.dot(ctx.astype(jnp.bfloat16), wo_ref[sl, :],
                                    preferred_element_type=jnp.float32)
            work_ref[b * SQ:(b + 1) * SQ, :] = acc.astype(jnp.bfloat16)

        for d in range(N_DEV):
            @pl.when(d != my)
            def _():
                pltpu.make_async_remote_copy(
                    src_ref=work_ref.at[pl.ds(d * CH, CH)],
                    dst_ref=land_ref.at[my],
                    send_sem=rs_send.at[d],
                    recv_sem=rs_recv.at[my],
                    device_id=(jnp.int32(d),),
                    device_id_type=pl.DeviceIdType.MESH,
                ).start()

        for j in range(N_DEV):
            @pl.when(j != my)
            def _():
                pltpu.make_async_remote_copy(
                    src_ref=work_ref.at[pl.ds(j * CH, CH)],
                    dst_ref=land_ref.at[j],
                    send_sem=rs_send.at[j],
                    recv_sem=rs_recv.at[j],
                    device_id=(jnp.int32(j),),
                    device_id_type=pl.DeviceIdType.MESH,
                ).wait_recv()

        red = work_ref[pl.ds(my * CH, CH), :].astype(jnp.float32)
        for j in range(N_DEV):
            red = red + jnp.where(
                j == my, jnp.float32(0),
                land_ref[j, :, :].astype(jnp.float32))
        work_ref[pl.ds(my * CH, CH), :] = red.astype(jnp.bfloat16)

        for d in range(N_DEV):
            @pl.when(d != my)
            def _():
                pltpu.make_async_remote_copy(
                    src_ref=work_ref.at[pl.ds(my * CH, CH)],
                    dst_ref=work_ref.at[pl.ds(my * CH, CH)],
                    send_sem=ag_send.at[d],
                    recv_sem=ag_recv.at[my],
                    device_id=(jnp.int32(d),),
                    device_id_type=pl.DeviceIdType.MESH,
                ).start()

        for j in range(N_DEV):
            @pl.when(j != my)
            def _():
                pltpu.make_async_remote_copy(
                    src_ref=work_ref.at[pl.ds(j * CH, CH)],
                    dst_ref=work_ref.at[pl.ds(j * CH, CH)],
                    send_sem=ag_send.at[j],
                    recv_sem=ag_recv.at[j],
                    device_id=(jnp.int32(j),),
                    device_id_type=pl.DeviceIdType.MESH,
                ).wait_recv()

        for b in range(B):
            out_ref[b, :, :] = work_ref[b * SQ:(b + 1) * SQ, :].astype(jnp.float32)

        for d in range(N_DEV):
            @pl.when(d != my)
            def _():
                pltpu.make_async_remote_copy(
                    src_ref=work_ref.at[pl.ds(d * CH, CH)],
                    dst_ref=land_ref.at[my],
                    send_sem=rs_send.at[d],
                    recv_sem=rs_recv.at[my],
                    device_id=(jnp.int32(d),),
                    device_id_type=pl.DeviceIdType.MESH,
                ).wait_send()
                pltpu.make_async_remote_copy(
                    src_ref=work_ref.at[pl.ds(my * CH, CH)],
                    dst_ref=work_ref.at[pl.ds(my * CH, CH)],
                    send_sem=ag_send.at[d],
                    recv_sem=ag_recv.at[my],
                    device_id=(jnp.int32(d),),
                    device_id_type=pl.DeviceIdType.MESH,
                ).wait_send()

    return pl.pallas_call(
        body,
        out_shape=jax.ShapeDtypeStruct((B, SQ, D), jnp.float32),
        in_specs=[pl.BlockSpec(memory_space=pltpu.VMEM)] * 7,
        out_specs=pl.BlockSpec(memory_space=pltpu.VMEM),
        scratch_shapes=[
            pltpu.VMEM((ROWS, D), jnp.bfloat16),
            pltpu.VMEM((N_DEV, CH, D), jnp.bfloat16),
            pltpu.SemaphoreType.DMA((N_DEV,)),
            pltpu.SemaphoreType.DMA((N_DEV,)),
            pltpu.SemaphoreType.DMA((N_DEV,)),
            pltpu.SemaphoreType.DMA((N_DEV,)),
        ],
        compiler_params=pltpu.CompilerParams(collective_id=0),
    )(xb, wq, wk, wv, wo, cos_d, sin_d)


# device time: 21895 ns/iter; 7.9333x vs baseline; 1.7207x over previous
import os

import numpy as np
import jax
import jax.numpy as jnp
from jax import lax
from jax.experimental import pallas as pl
from jax.experimental.pallas import tpu as pltpu

_SKIP_COMM = os.environ.get("SKIP_COMM") == "1"

N_DEV = 16
B, SQ, D = 2, 256, 768
ROWS = B * SQ
CH = ROWS // N_DEV
HQ_LOCAL, DH = 4, 64


def _rope_tables():
    inv = 1.0 / (10000.0 ** (np.arange(0, DH, 2) / DH))
    pos = np.arange(SQ)[:, None] * inv[None, :]
    cosv = np.cos(pos).astype(np.float32)
    sinv = np.sin(pos).astype(np.float32)
    cos_d = np.concatenate([cosv, cosv], axis=1)
    sin_d = np.concatenate([sinv, sinv], axis=1)
    return jnp.asarray(cos_d), jnp.asarray(sin_d)


def _deinterleave_cols(w):
    return w.reshape(D, HQ_LOCAL, DH // 2, 2).transpose(0, 1, 3, 2).reshape(D, HQ_LOCAL * DH)


def kernel(x, Wq, Wk, Wv, Wo):
    cos_d, sin_d = _rope_tables()
    xb = x.astype(jnp.bfloat16)
    wq = _deinterleave_cols(Wq).astype(jnp.bfloat16)
    wk = _deinterleave_cols(Wk).astype(jnp.bfloat16)
    wv = Wv.astype(jnp.bfloat16)
    wo = Wo.astype(jnp.bfloat16)

    def body(x_ref, wq_ref, wk_ref, wv_ref, wo_ref, cos_ref, sin_ref,
             out_ref, work_ref, land_ref,
             rs_send, rs_recv, ag_send, ag_recv):
        my = lax.axis_index("i")

        barrier_sem = pltpu.get_barrier_semaphore()
        for j in range(N_DEV):
            @pl.when(j != my)
            def _():
                pl.semaphore_signal(
                    barrier_sem, inc=1,
                    device_id=(jnp.int32(j),),
                    device_id_type=pl.DeviceIdType.MESH,
                )
        pl.semaphore_wait(barrier_sem, N_DEV - 1)

        cos = cos_ref[...]
        sin = sin_ref[...]
        for b in range(B):
            x_b = x_ref[b, :, :]
            acc = jnp.zeros((SQ, D), dtype=jnp.float32)
            for h in range(HQ_LOCAL):
                sl = slice(h * DH, (h + 1) * DH)
                q = jnp.dot(x_b, wq_ref[:, sl],
                            preferred_element_type=jnp.float32)
                k = jnp.dot(x_b, wk_ref[:, sl],
                            preferred_element_type=jnp.float32)
                v = jnp.dot(x_b, wv_ref[:, sl],
                            preferred_element_type=jnp.float32)

                def rope(t):
                    t_r = jnp.concatenate(
                        [-t[:, DH // 2:], t[:, :DH // 2]], axis=1)
                    return t * cos + t_r * sin

                q = rope(q).astype(jnp.bfloat16)
                k = rope(k).astype(jnp.bfloat16)
                s_ = lax.dot_general(
                    q, k, (((1,), (1,)), ((), ())),
                    preferred_element_type=jnp.float32) * 0.125
                m = jnp.max(s_, axis=1, keepdims=True)
                e = jnp.exp(s_ - m)
                w = (e / jnp.sum(e, axis=1, keepdims=True)).astype(jnp.bfloat16)
                ctx = jnp.dot(w, v.astype(jnp.bfloat16),
                              preferred_element_type=jnp.float32)
                acc = acc + jnp.dot(ctx.astype(jnp.bfloat16), wo_ref[sl, :],
                                    preferred_element_type=jnp.float32)
            work_ref[b * SQ:(b + 1) * SQ, :] = acc.astype(jnp.bfloat16)

        if _SKIP_COMM:
            for b in range(B):
                out_ref[b, :, :] = work_ref[b * SQ:(b + 1) * SQ, :].astype(jnp.float32)
            return

        for d in range(N_DEV):
            @pl.when(d != my)
            def _():
                pltpu.make_async_remote_copy(
                    src_ref=work_ref.at[pl.ds(d * CH, CH)],
                    dst_ref=land_ref.at[my],
                    send_sem=rs_send.at[d],
                    recv_sem=rs_recv.at[my],
                    device_id=(jnp.int32(d),),
                    device_id_type=pl.DeviceIdType.MESH,
                ).start()

        for j in range(N_DEV):
            @pl.when(j != my)
            def _():
                pltpu.make_async_remote_copy(
                    src_ref=work_ref.at[pl.ds(j * CH, CH)],
                    dst_ref=land_ref.at[j],
                    send_sem=rs_send.at[j],
                    recv_sem=rs_recv.at[j],
                    device_id=(jnp.int32(j),),
                    device_id_type=pl.DeviceIdType.MESH,
                ).wait_recv()

        red = work_ref[pl.ds(my * CH, CH), :].astype(jnp.float32)
        for j in range(N_DEV):
            red = red + jnp.where(
                j == my, jnp.float32(0),
                land_ref[j, :, :].astype(jnp.float32))
        work_ref[pl.ds(my * CH, CH), :] = red.astype(jnp.bfloat16)

        for d in range(N_DEV):
            @pl.when(d != my)
            def _():
                pltpu.make_async_remote_copy(
                    src_ref=work_ref.at[pl.ds(my * CH, CH)],
                    dst_ref=work_ref.at[pl.ds(my * CH, CH)],
                    send_sem=ag_send.at[d],
                    recv_sem=ag_recv.at[my],
                    device_id=(jnp.int32(d),),
                    device_id_type=pl.DeviceIdType.MESH,
                ).start()

        for j in range(N_DEV):
            @pl.when(j != my)
            def _():
                pltpu.make_async_remote_copy(
                    src_ref=work_ref.at[pl.ds(j * CH, CH)],
                    dst_ref=work_ref.at[pl.ds(j * CH, CH)],
                    send_sem=ag_send.at[j],
                    recv_sem=ag_recv.at[j],
                    device_id=(jnp.int32(j),),
                    device_id_type=pl.DeviceIdType.MESH,
                ).wait_recv()

        for b in range(B):
            out_ref[b, :, :] = work_ref[b * SQ:(b + 1) * SQ, :].astype(jnp.float32)

        for d in range(N_DEV):
            @pl.when(d != my)
            def _():
                pltpu.make_async_remote_copy(
                    src_ref=work_ref.at[pl.ds(d * CH, CH)],
                    dst_ref=land_ref.at[my],
                    send_sem=rs_send.at[d],
                    recv_sem=rs_recv.at[my],
                    device_id=(jnp.int32(d),),
                    device_id_type=pl.DeviceIdType.MESH,
                ).wait_send()
                pltpu.make_async_remote_copy(
                    src_ref=work_ref.at[pl.ds(my * CH, CH)],
                    dst_ref=work_ref.at[pl.ds(my * CH, CH)],
                    send_sem=ag_send.at[d],
                    recv_sem=ag_recv.at[my],
                    device_id=(jnp.int32(d),),
                    device_id_type=pl.DeviceIdType.MESH,
                ).wait_send()

    return pl.pallas_call(
        body,
        out_shape=jax.ShapeDtypeStruct((B, SQ, D), jnp.float32),
        in_specs=[pl.BlockSpec(memory_space=pltpu.VMEM)] * 7,
        out_specs=pl.BlockSpec(memory_space=pltpu.VMEM),
        scratch_shapes=[
            pltpu.VMEM((ROWS, D), jnp.bfloat16),
            pltpu.VMEM((N_DEV, CH, D), jnp.bfloat16),
            pltpu.SemaphoreType.DMA((N_DEV,)),
            pltpu.SemaphoreType.DMA((N_DEV,)),
            pltpu.SemaphoreType.DMA((N_DEV,)),
            pltpu.SemaphoreType.DMA((N_DEV,)),
        ],
        compiler_params=pltpu.CompilerParams(collective_id=0),
    )(xb, wq, wk, wv, wo, cos_d, sin_d)


# device time: 10931 ns/iter; 15.8906x vs baseline; 2.0030x over previous
import os

import numpy as np
import jax
import jax.numpy as jnp
from jax import lax
from jax.experimental import pallas as pl
from jax.experimental.pallas import tpu as pltpu

_SKIP_COMM = os.environ.get("SKIP_COMM") == "1"

N_DEV = 16
B, SQ, D = 2, 256, 768
ROWS = B * SQ
CH = ROWS // N_DEV
HQ_LOCAL, DH = 4, 64
HD = HQ_LOCAL * DH


def _rope_tables():
    inv = 1.0 / (10000.0 ** (np.arange(0, DH, 2) / DH))
    pos = np.arange(SQ)[:, None] * inv[None, :]
    cosv = np.cos(pos).astype(np.float32)
    sinv = np.sin(pos).astype(np.float32)
    cos_d = np.tile(np.concatenate([cosv, cosv], axis=1), (1, HQ_LOCAL))
    sin_d = np.tile(np.concatenate([sinv, sinv], axis=1), (1, HQ_LOCAL))
    return jnp.asarray(cos_d), jnp.asarray(sin_d)


def _deinterleave_cols(w):
    return w.reshape(D, HQ_LOCAL, DH // 2, 2).transpose(0, 1, 3, 2).reshape(D, HD)


def kernel(x, Wq, Wk, Wv, Wo):
    cos_d, sin_d = _rope_tables()
    xb = x.astype(jnp.bfloat16)
    wq = _deinterleave_cols(Wq).astype(jnp.bfloat16)
    wk = _deinterleave_cols(Wk).astype(jnp.bfloat16)
    wv = Wv.astype(jnp.bfloat16)
    wo = Wo.astype(jnp.bfloat16)

    def body(x_ref, wq_ref, wk_ref, wv_ref, wo_ref, cos_ref, sin_ref,
             out_ref, work_ref, land_ref,
             rs_send, rs_recv, ag_send, ag_recv):
        my = lax.axis_index("i")

        barrier_sem = pltpu.get_barrier_semaphore()
        for j in range(N_DEV):
            @pl.when(j != my)
            def _():
                pl.semaphore_signal(
                    barrier_sem, inc=1,
                    device_id=(jnp.int32(j),),
                    device_id_type=pl.DeviceIdType.MESH,
                )
        pl.semaphore_wait(barrier_sem, N_DEV - 1)

        def rs_send_chunk(d):
            @pl.when(d != my)
            def _():
                pltpu.make_async_remote_copy(
                    src_ref=work_ref.at[pl.ds(d * CH, CH)],
                    dst_ref=land_ref.at[my],
                    send_sem=rs_send.at[d],
                    recv_sem=rs_recv.at[my],
                    device_id=(jnp.int32(d),),
                    device_id_type=pl.DeviceIdType.MESH,
                ).start()

        cos = cos_ref[...]
        sin = sin_ref[...]
        for b in range(B):
            x_b = x_ref[b, :, :]
            q = jnp.dot(x_b, wq_ref[...], preferred_element_type=jnp.float32)
            k = jnp.dot(x_b, wk_ref[...], preferred_element_type=jnp.float32)
            v = jnp.dot(x_b, wv_ref[...],
                        preferred_element_type=jnp.float32).astype(jnp.bfloat16)

            def rope(t):
                parts = []
                for h in range(HQ_LOCAL):
                    lo, mid, hi = h * DH, h * DH + DH // 2, (h + 1) * DH
                    parts.append(-t[:, mid:hi])
                    parts.append(t[:, lo:mid])
                return t * cos + jnp.concatenate(parts, axis=1) * sin

            q = rope(q).astype(jnp.bfloat16)
            k = rope(k).astype(jnp.bfloat16)

            ctxs = []
            for h in range(HQ_LOCAL):
                sl = slice(h * DH, (h + 1) * DH)
                s_ = lax.dot_general(
                    q[:, sl], k[:, sl], (((1,), (1,)), ((), ())),
                    preferred_element_type=jnp.float32) * 0.125
                m = jnp.max(s_, axis=1, keepdims=True)
                e = jnp.exp(s_ - m)
                w = (e / jnp.sum(e, axis=1, keepdims=True)).astype(jnp.bfloat16)
                ctxs.append(jnp.dot(w, v[:, sl],
                                    preferred_element_type=jnp.float32)
                            .astype(jnp.bfloat16))
            ctx = jnp.concatenate(ctxs, axis=1)
            part = jnp.dot(ctx, wo_ref[...], preferred_element_type=jnp.float32)
            work_ref[b * SQ:(b + 1) * SQ, :] = part.astype(jnp.bfloat16)

            if not _SKIP_COMM:
                for d in range(b * N_DEV // B, (b + 1) * N_DEV // B):
                    rs_send_chunk(d)

        if _SKIP_COMM:
            for b in range(B):
                out_ref[b, :, :] = work_ref[b * SQ:(b + 1) * SQ, :].astype(jnp.float32)
            return

        for j in range(N_DEV):
            @pl.when(j != my)
            def _():
                pltpu.make_async_remote_copy(
                    src_ref=work_ref.at[pl.ds(j * CH, CH)],
                    dst_ref=land_ref.at[j],
                    send_sem=rs_send.at[j],
                    recv_sem=rs_recv.at[j],
                    device_id=(jnp.int32(j),),
                    device_id_type=pl.DeviceIdType.MESH,
                ).wait_recv()

        red = work_ref[pl.ds(my * CH, CH), :].astype(jnp.float32)
        for j in range(N_DEV):
            red = red + jnp.where(
                j == my, jnp.float32(0),
                land_ref[j, :, :].astype(jnp.float32))
        work_ref[pl.ds(my * CH, CH), :] = red.astype(jnp.bfloat16)

        for d in range(N_DEV):
            @pl.when(d != my)
            def _():
                pltpu.make_async_remote_copy(
                    src_ref=work_ref.at[pl.ds(my * CH, CH)],
                    dst_ref=work_ref.at[pl.ds(my * CH, CH)],
                    send_sem=ag_send.at[d],
                    recv_sem=ag_recv.at[my],
                    device_id=(jnp.int32(d),),
                    device_id_type=pl.DeviceIdType.MESH,
                ).start()

        for j in range(N_DEV):
            @pl.when(j != my)
            def _():
                pltpu.make_async_remote_copy(
                    src_ref=work_ref.at[pl.ds(j * CH, CH)],
                    dst_ref=work_ref.at[pl.ds(j * CH, CH)],
                    send_sem=ag_send.at[j],
                    recv_sem=ag_recv.at[j],
                    device_id=(jnp.int32(j),),
                    device_id_type=pl.DeviceIdType.MESH,
                ).wait_recv()

        for b in range(B):
            out_ref[b, :, :] = work_ref[b * SQ:(b + 1) * SQ, :].astype(jnp.float32)

        for d in range(N_DEV):
            @pl.when(d != my)
            def _():
                pltpu.make_async_remote_copy(
                    src_ref=work_ref.at[pl.ds(d * CH, CH)],
                    dst_ref=land_ref.at[my],
                    send_sem=rs_send.at[d],
                    recv_sem=rs_recv.at[my],
                    device_id=(jnp.int32(d),),
                    device_id_type=pl.DeviceIdType.MESH,
                ).wait_send()
                pltpu.make_async_remote_copy(
                    src_ref=work_ref.at[pl.ds(my * CH, CH)],
                    dst_ref=work_ref.at[pl.ds(my * CH, CH)],
                    send_sem=ag_send.at[d],
                    recv_sem=ag_recv.at[my],
                    device_id=(jnp.int32(d),),
                    device_id_type=pl.DeviceIdType.MESH,
                ).wait_send()

    return pl.pallas_call(
        body,
        out_shape=jax.ShapeDtypeStruct((B, SQ, D), jnp.float32),
        in_specs=[pl.BlockSpec(memory_space=pltpu.VMEM)] * 7,
        out_specs=pl.BlockSpec(memory_space=pltpu.VMEM),
        scratch_shapes=[
            pltpu.VMEM((ROWS, D), jnp.bfloat16),
            pltpu.VMEM((N_DEV, CH, D), jnp.bfloat16),
            pltpu.SemaphoreType.DMA((N_DEV,)),
            pltpu.SemaphoreType.DMA((N_DEV,)),
            pltpu.SemaphoreType.DMA((N_DEV,)),
            pltpu.SemaphoreType.DMA((N_DEV,)),
        ],
        compiler_params=pltpu.CompilerParams(collective_id=0),
    )(xb, wq, wk, wv, wo, cos_d, sin_d)
